# baseline (speedup 1.0000x reference)
import functools

import jax
import jax.numpy as jnp
from jax.experimental import pallas as pl
from jax.experimental.pallas import tpu as pltpu

_KT = 512


def _body(a_ref, p_ref, wte_ref, wpe_ref, b_ref, o_ref, *, kt, v):
    k = pl.program_id(0)

    @pl.when(k == 0)
    def _init():
        p = p_ref[...].astype(jnp.bfloat16)
        wp = wpe_ref[...].astype(jnp.bfloat16)
        acc = jax.lax.dot_general(
            p, wp, (((1,), (1,)), ((), ())), preferred_element_type=jnp.float32
        )
        o_ref[...] = acc + b_ref[...]

    valid = v - k * kt
    a = a_ref[...]
    w = wte_ref[...]
    a = jnp.where(jax.lax.broadcasted_iota(jnp.int32, a.shape, 1) < valid, a, 0)
    w = jnp.where(jax.lax.broadcasted_iota(jnp.int32, w.shape, 1) < valid, w, 0)
    o_ref[...] += jax.lax.dot_general(
        a.astype(jnp.bfloat16),
        w.astype(jnp.bfloat16),
        (((1,), (1,)), ((), ())),
        preferred_element_type=jnp.float32,
    )


def kernel(input_ids, position_ids, W_wte, b_wte, W_wpe, b_wpe):
    b, s, v = input_ids.shape
    d = W_wte.shape[0]
    npos = position_ids.shape[-1]
    m = b * s
    a2 = input_ids.reshape(m, v)
    p2 = position_ids.reshape(m, npos)
    bias = (b_wte + b_wpe).reshape(1, d)
    nk = pl.cdiv(v, _KT)
    out = pl.pallas_call(
        functools.partial(_body, kt=_KT, v=v),
        grid=(nk,),
        in_specs=[
            pl.BlockSpec((m, _KT), lambda k: (0, k)),
            pl.BlockSpec((m, npos), lambda k: (0, 0)),
            pl.BlockSpec((d, _KT), lambda k: (0, k)),
            pl.BlockSpec((d, npos), lambda k: (0, 0)),
            pl.BlockSpec((1, d), lambda k: (0, 0)),
        ],
        out_specs=pl.BlockSpec((m, d), lambda k: (0, 0)),
        out_shape=jax.ShapeDtypeStruct((m, d), jnp.float32),
        compiler_params=pltpu.CompilerParams(
            dimension_semantics=("arbitrary",)
        ),
    )(a2, p2, W_wte, W_wpe, bias)
    return out.reshape(b, s, d)


# speedup vs baseline: 1.0077x; 1.0077x over previous
import functools

import jax
import jax.numpy as jnp
from jax.experimental import pallas as pl
from jax.experimental.pallas import tpu as pltpu

_KT = 512


def _dot_t(x, y):
    return jax.lax.dot_general(
        x, y, (((1,), (1,)), ((), ())), preferred_element_type=jnp.float32
    )


def _body(a_ref, p_ref, wte_ref, wpe_ref, b_ref, o_ref, *, kt, v):
    k = pl.program_id(0)
    nk = pl.num_programs(0)

    @pl.when(k == 0)
    def _init():
        o_ref[...] = _dot_t(p_ref[...], wpe_ref[...]) + b_ref[...]

    @pl.when(k < nk - 1)
    def _full():
        o_ref[...] += _dot_t(a_ref[...], wte_ref[...])

    @pl.when(k == nk - 1)
    def _tail():
        valid = v - k * kt
        a = a_ref[...]
        w = wte_ref[...]
        a = jnp.where(
            jax.lax.broadcasted_iota(jnp.int32, a.shape, 1) < valid, a, 0
        )
        w = jnp.where(
            jax.lax.broadcasted_iota(jnp.int32, w.shape, 1) < valid, w, 0
        )
        o_ref[...] += _dot_t(a, w)


def kernel(input_ids, position_ids, W_wte, b_wte, W_wpe, b_wpe):
    b, s, v = input_ids.shape
    d = W_wte.shape[0]
    npos = position_ids.shape[-1]
    m = b * s
    a2 = input_ids.reshape(m, v)
    p2 = position_ids.reshape(m, npos)
    bias = (b_wte + b_wpe).reshape(1, d)
    nk = pl.cdiv(v, _KT)
    out = pl.pallas_call(
        functools.partial(_body, kt=_KT, v=v),
        grid=(nk,),
        in_specs=[
            pl.BlockSpec((m, _KT), lambda k: (0, k)),
            pl.BlockSpec((m, npos), lambda k: (0, 0)),
            pl.BlockSpec((d, _KT), lambda k: (0, k)),
            pl.BlockSpec((d, npos), lambda k: (0, 0)),
            pl.BlockSpec((1, d), lambda k: (0, 0)),
        ],
        out_specs=pl.BlockSpec((m, d), lambda k: (0, 0)),
        out_shape=jax.ShapeDtypeStruct((m, d), jnp.float32),
        compiler_params=pltpu.CompilerParams(
            dimension_semantics=("arbitrary",)
        ),
    )(a2, p2, W_wte, W_wpe, bias)
    return out.reshape(b, s, d)


# speedup vs baseline: 1.0117x; 1.0040x over previous
import functools

import jax
import jax.numpy as jnp
from jax.experimental import pallas as pl
from jax.experimental.pallas import tpu as pltpu

_KT = 512


def _dot_t(x, y):
    return jax.lax.dot_general(
        x, y, (((1,), (1,)), ((), ())), preferred_element_type=jnp.float32
    )


def _body(a_ref, p_ref, wte_ref, wpe_ref, b_ref, o_ref, *, kt, v):
    k = pl.program_id(0)
    nk = pl.num_programs(0)

    @pl.when(k == 0)
    def _init():
        o_ref[...] = (
            _dot_t(
                p_ref[...].astype(jnp.bfloat16),
                wpe_ref[...].astype(jnp.bfloat16),
            )
            + b_ref[...]
        )

    @pl.when(k < nk - 1)
    def _full():
        o_ref[...] += _dot_t(
            a_ref[...].astype(jnp.bfloat16), wte_ref[...].astype(jnp.bfloat16)
        )

    @pl.when(k == nk - 1)
    def _tail():
        valid = v - k * kt
        a = a_ref[...]
        w = wte_ref[...]
        a = jnp.where(
            jax.lax.broadcasted_iota(jnp.int32, a.shape, 1) < valid, a, 0
        )
        w = jnp.where(
            jax.lax.broadcasted_iota(jnp.int32, w.shape, 1) < valid, w, 0
        )
        o_ref[...] += _dot_t(a.astype(jnp.bfloat16), w.astype(jnp.bfloat16))


def kernel(input_ids, position_ids, W_wte, b_wte, W_wpe, b_wpe):
    b, s, v = input_ids.shape
    d = W_wte.shape[0]
    npos = position_ids.shape[-1]
    m = b * s
    a2 = input_ids.reshape(m, v)
    p2 = position_ids.reshape(m, npos)
    bias = (b_wte + b_wpe).reshape(1, d)
    nk = pl.cdiv(v, _KT)
    out = pl.pallas_call(
        functools.partial(_body, kt=_KT, v=v),
        grid=(nk,),
        in_specs=[
            pl.BlockSpec((m, _KT), lambda k: (0, k)),
            pl.BlockSpec((m, npos), lambda k: (0, 0)),
            pl.BlockSpec((d, _KT), lambda k: (0, k)),
            pl.BlockSpec((d, npos), lambda k: (0, 0)),
            pl.BlockSpec((1, d), lambda k: (0, 0)),
        ],
        out_specs=pl.BlockSpec((m, d), lambda k: (0, 0)),
        out_shape=jax.ShapeDtypeStruct((m, d), jnp.float32),
        compiler_params=pltpu.CompilerParams(
            dimension_semantics=("arbitrary",)
        ),
    )(a2, p2, W_wte, W_wpe, bias)
    return out.reshape(b, s, d)


# speedup vs baseline: 1.3539x; 1.3382x over previous
import functools

import jax
import jax.numpy as jnp
from jax.experimental import pallas as pl
from jax.experimental.pallas import tpu as pltpu

_KT = 512
_LANE = 128


def _dot_k0(x, y):
    return jax.lax.dot_general(
        x, y, (((0,), (0,)), ((), ())), preferred_element_type=jnp.float32
    )


def _body(a3_ref, p_ref, wt_ref, wpe_ref, b_ref, o_ref, *, kt, v, sgrp):
    k = pl.program_id(0)
    nk = pl.num_programs(0)

    @pl.when(k == 0)
    def _init():
        p = p_ref[...].astype(jnp.bfloat16)
        wp = wpe_ref[...].astype(jnp.bfloat16)
        acc = jax.lax.dot_general(
            p, wp, (((1,), (1,)), ((), ())), preferred_element_type=jnp.float32
        )
        o_ref[...] = acc + b_ref[...]

    valid = v - k * kt

    @pl.when(k < nk - 1)
    def _full():
        w = wt_ref[...].astype(jnp.bfloat16)
        for i in range(sgrp):
            a = a3_ref[:, i, :].astype(jnp.bfloat16)
            o_ref[pl.ds(i * _LANE, _LANE), :] += _dot_k0(a, w)

    @pl.when(k == nk - 1)
    def _tail():
        w = wt_ref[...]
        w = jnp.where(
            jax.lax.broadcasted_iota(jnp.int32, w.shape, 0) < valid, w, 0
        ).astype(jnp.bfloat16)
        for i in range(sgrp):
            a = a3_ref[:, i, :]
            a = jnp.where(
                jax.lax.broadcasted_iota(jnp.int32, a.shape, 0) < valid, a, 0
            ).astype(jnp.bfloat16)
            o_ref[pl.ds(i * _LANE, _LANE), :] += _dot_k0(a, w)


def kernel(input_ids, position_ids, W_wte, b_wte, W_wpe, b_wpe):
    b, s, v = input_ids.shape
    d = W_wte.shape[0]
    npos = position_ids.shape[-1]
    m = b * s
    sgrp = m // _LANE
    a3 = jnp.transpose(input_ids, (2, 0, 1)).reshape(v, sgrp, _LANE)
    wt = jnp.transpose(W_wte)
    p2 = position_ids.reshape(m, npos)
    bias = (b_wte + b_wpe).reshape(1, d)
    nk = pl.cdiv(v, _KT)
    out = pl.pallas_call(
        functools.partial(_body, kt=_KT, v=v, sgrp=sgrp),
        grid=(nk,),
        in_specs=[
            pl.BlockSpec((_KT, sgrp, _LANE), lambda k: (k, 0, 0)),
            pl.BlockSpec((m, npos), lambda k: (0, 0)),
            pl.BlockSpec((_KT, d), lambda k: (k, 0)),
            pl.BlockSpec((d, npos), lambda k: (0, 0)),
            pl.BlockSpec((1, d), lambda k: (0, 0)),
        ],
        out_specs=pl.BlockSpec((m, d), lambda k: (0, 0)),
        out_shape=jax.ShapeDtypeStruct((m, d), jnp.float32),
        compiler_params=pltpu.CompilerParams(
            dimension_semantics=("arbitrary",)
        ),
    )(a3, p2, wt, W_wpe, bias)
    return out.reshape(b, s, d)
